# Initial kernel scaffold; baseline (speedup 1.0000x reference)
import jax
import jax.numpy as jnp
from jax.experimental import pallas as pl


def kernel(node_tensors, edge_tensors, edge_index, params):
    raise NotImplementedError("write your pallas kernel here")


# speedup vs baseline: 2.7783x; 2.7783x over previous
import functools

import jax
import jax.numpy as jnp
import numpy as np
from jax import lax
from jax.experimental import pallas as pl
from jax.experimental.pallas import tpu as pltpu
from jax.experimental.pallas import tpu_sc as plsc

N = 1024
E = 4096
D = 128
H = 8
DH = D // H
HID = 256
SCALE = 1.0 / np.sqrt(DH)
NEG = -1e30

EB = 512
NB = 256
CB = 1024


def _sc_gather(table, idx):
    B = idx.shape[0]
    Dt = table.shape[1]
    info = plsc.get_sparse_core_info()
    nc = info.num_cores
    nw = nc * info.num_subcores
    b_per_w = B // nw
    chunk = min(b_per_w, 128)
    n_chunks = b_per_w // chunk
    mesh = plsc.VectorSubcoreMesh(core_axis_name="c", subcore_axis_name="s")

    @functools.partial(
        pl.kernel, mesh=mesh,
        out_type=jax.ShapeDtypeStruct((B, Dt), jnp.float32),
        scratch_types=[
            pltpu.VMEM((n_chunks, chunk), jnp.int32),
            pltpu.VMEM((b_per_w, Dt), jnp.float32),
            pltpu.SemaphoreType.DMA,
        ],
    )
    def k(table_hbm, idx_hbm, out_hbm, idx_v, rows_v, sem):
        wid = lax.axis_index("s") * nc + lax.axis_index("c")
        base = wid * b_per_w
        for j in range(n_chunks):
            pltpu.sync_copy(idx_hbm.at[pl.ds(base + j * chunk, chunk)],
                            idx_v.at[j])
            pltpu.async_copy(table_hbm.at[idx_v.at[j]],
                             rows_v.at[pl.ds(j * chunk, chunk)], sem).wait()
        pltpu.sync_copy(rows_v, out_hbm.at[pl.ds(base, b_per_w)])

    return k(table, idx)


def _ln(x, g, b):
    m = jnp.mean(x, axis=-1, keepdims=True)
    v = jnp.mean((x - m) ** 2, axis=-1, keepdims=True)
    return (x - m) / jnp.sqrt(v + 1e-5) * g + b


def _dot(a, b):
    return jnp.dot(a, b, preferred_element_type=jnp.float32)


def _rep_heads(x, rows):
    return jnp.concatenate(
        [jnp.broadcast_to(x[:, h:h + 1], (rows, DH)) for h in range(H)], axis=1)


def _nprep_body(et_ref, g0_ref, g1_ref, lne_g, lne_b, lnn_g, lnn_b,
                weqkv, beqkv, wnq, bnq, wnkv, bnkv, s_ref, ve_ref):
    xe = _ln(et_ref[...], lne_g[...], lne_b[...])
    eqkv = _dot(xe, weqkv[...]) + beqkv[...]
    x0 = _ln(g0_ref[...], lnn_g[...], lnn_b[...])
    x1 = _ln(g1_ref[...], lnn_g[...], lnn_b[...])
    q = eqkv[:, :D] + _dot(x0, wnq[...]) + bnq[...]
    kv = eqkv[:, D:] + _dot(x1, wnkv[...]) + bnkv[...]
    ve_ref[...] = kv[:, D:]
    qk = q * kv[:, :D]
    cols = [jnp.sum(qk[:, h * DH:(h + 1) * DH], axis=1, keepdims=True)
            for h in range(H)]
    s_ref[...] = jnp.concatenate(cols, axis=1) * SCALE


def _nagg_body(s_ref, ve_ref, ei0_ref, nt_ref, wout, bout, lnf_g, lnf_b,
               fc1, bfc1, fc2, bfc2, out_ref):
    i = pl.program_id(0)
    s = s_ref[...]
    m = jnp.max(s, axis=0, keepdims=True)
    e = jnp.exp(s - m)
    ev = _rep_heads(e, E) * ve_ref[...]
    rows = lax.broadcasted_iota(jnp.int32, (NB, E), 0) + (i * NB)
    oh = jnp.where(rows == ei0_ref[...], 1.0, 0.0)
    den = _dot(oh, e)
    acc = _dot(oh, ev)
    upd = acc / (_rep_heads(den, NB) + 1e-16)
    nt1 = _dot(upd, wout[...]) + bout[...] + nt_ref[...]
    x = _ln(nt1, lnf_g[...], lnf_b[...])
    h1 = jnp.maximum(_dot(x, fc1[...]) + bfc1[...], 0.0)
    out_ref[...] = _dot(h1, fc2[...]) + bfc2[...] + nt1


def _eqkv_body(et_ref, g2_ref, g3_ref, lne_g, lne_b, lnn_g, lnn_b,
               weqkv, beqkv, wnqkv, bnqkv, out_ref):
    xe = _ln(et_ref[...], lne_g[...], lne_b[...])
    base = _dot(xe, weqkv[...]) + beqkv[...]
    x2 = _ln(g2_ref[...], lnn_g[...], lnn_b[...])
    x3 = _ln(g3_ref[...], lnn_g[...], lnn_b[...])
    out_ref[...] = base + _dot(x2 + x3, wnqkv[...]) + 2.0 * bnqkv[...]


def _eattn_body(qkv_blk, kv_chunk, ei0r, ei1r, ei0c, ei1c, et_ref,
                wout, bout, lnf_g, lnf_b, fc1, bfc1, fc2, bfc2, out_ref,
                m_s, den_s, acc_s):
    c = pl.program_id(1)
    nc = pl.num_programs(1)

    @pl.when(c == 0)
    def _init():
        m_s[...] = jnp.zeros((EB, H), jnp.float32)
        den_s[...] = jnp.zeros((EB, H), jnp.float32)
        acc_s[...] = jnp.zeros((EB, D), jnp.float32)

    qall = qkv_blk[...][:, :D]
    kc = kv_chunk[...][:, D:2 * D]
    vc = kv_chunk[...][:, 2 * D:]
    a0 = ei0c[...]
    a1 = ei1c[...]
    b0 = ei0r[...]
    b1 = ei1r[...]
    nb = (a0 == b0) | (a0 == b1) | (a1 == b0) | (a1 == b1)
    neg = jnp.where(nb, 0.0, NEG)
    for h in range(H):
        hs = slice(h * DH, (h + 1) * DH)
        sh = lax.dot_general(qall[:, hs], kc[:, hs], (((1,), (1,)), ((), ())),
                             preferred_element_type=jnp.float32)
        sh = sh * SCALE + neg
        mo = m_s[:, h:h + 1]
        mn = jnp.maximum(mo, jnp.max(sh, axis=1, keepdims=True))
        alpha = jnp.exp(mo - mn)
        p = jnp.exp(sh - mn)
        den_s[:, h:h + 1] = den_s[:, h:h + 1] * alpha \
            + jnp.sum(p, axis=1, keepdims=True)
        acc_s[:, hs] = acc_s[:, hs] * alpha + _dot(p, vc[:, hs])
        m_s[:, h:h + 1] = mn

    @pl.when(c == nc - 1)
    def _epilogue():
        upd = acc_s[...] / (_rep_heads(den_s[...], EB) + 1e-16)
        et1 = _dot(upd, wout[...]) + bout[...] + et_ref[...]
        x = _ln(et1, lnf_g[...], lnf_b[...])
        h1 = jnp.maximum(_dot(x, fc1[...]) + bfc1[...], 0.0)
        out_ref[...] = _dot(h1, fc2[...]) + bfc2[...] + et1


def _row(v):
    return v.reshape(1, -1)


def _full(shape):
    return pl.BlockSpec(shape, lambda i: (0, 0))


def _eblk(cols=D):
    return pl.BlockSpec((EB, cols), lambda i: (i, 0))


def kernel(node_tensors, edge_tensors, edge_index, params):
    p = params
    nt = node_tensors
    et = edge_tensors
    ei0 = edge_index[0]
    ei1 = edge_index[1]
    idx = jnp.concatenate([ei0, ei1])
    ei0i = _row(ei0)
    ei0r = _row(ei0.astype(jnp.float32))
    ei1r = _row(ei1.astype(jnp.float32))
    ei0c = ei0.astype(jnp.float32).reshape(E, 1)
    ei1c = ei1.astype(jnp.float32).reshape(E, 1)

    nm = p["node_mha"]
    em = p["edge_mha"]
    weqkv_n = jnp.concatenate(
        [nm["Weq"]["W"].T, nm["Wek"]["W"].T, nm["Wev"]["W"].T], axis=1)
    beqkv_n = _row(jnp.concatenate(
        [nm["Weq"]["b"], nm["Wek"]["b"], nm["Wev"]["b"]]))
    wnq_n = nm["Wnq"]["W"].T
    bnq_n = _row(nm["Wnq"]["b"])
    wnkv_n = jnp.concatenate([nm["Wnk"]["W"].T, nm["Wnv"]["W"].T], axis=1)
    bnkv_n = _row(jnp.concatenate([nm["Wnk"]["b"], nm["Wnv"]["b"]]))
    wout_n = nm["out"]["W"].T
    bout_n = _row(nm["out"]["b"])
    weqkv_e = jnp.concatenate(
        [em["Weq"]["W"].T, em["Wek"]["W"].T, em["Wev"]["W"].T], axis=1)
    beqkv_e = _row(jnp.concatenate(
        [em["Weq"]["b"], em["Wek"]["b"], em["Wev"]["b"]]))
    wnqkv_e = jnp.concatenate(
        [em["Wnq"]["W"].T, em["Wnk"]["W"].T, em["Wnv"]["W"].T], axis=1)
    bnqkv_e = _row(jnp.concatenate(
        [em["Wnq"]["b"], em["Wnk"]["b"], em["Wnv"]["b"]]))
    wout_e = em["out"]["W"].T
    bout_e = _row(em["out"]["b"])
    nfc1 = p["node_ffn_fc1"]["W"].T
    nbfc1 = _row(p["node_ffn_fc1"]["b"])
    nfc2 = p["node_ffn_fc2"]["W"].T
    nbfc2 = _row(p["node_ffn_fc2"]["b"])
    efc1 = p["edge_ffn_fc1"]["W"].T
    ebfc1 = _row(p["edge_ffn_fc1"]["b"])
    efc2 = p["edge_ffn_fc2"]["W"].T
    ebfc2 = _row(p["edge_ffn_fc2"]["b"])
    ln = {k: (_row(v["g"]), _row(v["b"]))
          for k, v in p.items() if k.endswith("ln") or "_ln" in k}

    g01 = _sc_gather(nt, idx)
    g0 = g01[:E]
    g1 = g01[E:]

    wspec = lambda a: _full(a.shape)
    lnn1 = ln["node_attn_ln1"]
    lne1 = ln["edge_attn_ln1"]
    s, ve = pl.pallas_call(
        _nprep_body,
        grid=(E // EB,),
        in_specs=[_eblk(), _eblk(), _eblk(),
                  wspec(lne1[0]), wspec(lne1[1]), wspec(lnn1[0]), wspec(lnn1[1]),
                  wspec(weqkv_n), wspec(beqkv_n), wspec(wnq_n), wspec(bnq_n),
                  wspec(wnkv_n), wspec(bnkv_n)],
        out_specs=[_eblk(H), _eblk(D)],
        out_shape=[jax.ShapeDtypeStruct((E, H), jnp.float32),
                   jax.ShapeDtypeStruct((E, D), jnp.float32)],
    )(et, g0, g1, lne1[0], lne1[1], lnn1[0], lnn1[1],
      weqkv_n, beqkv_n, wnq_n, bnq_n, wnkv_n, bnkv_n)

    lnf = ln["node_ffn_ln"]
    nt2 = pl.pallas_call(
        _nagg_body,
        grid=(N // NB,),
        in_specs=[_full((E, H)), _full((E, D)), _full((1, E)),
                  pl.BlockSpec((NB, D), lambda i: (i, 0)),
                  wspec(wout_n), wspec(bout_n), wspec(lnf[0]), wspec(lnf[1]),
                  wspec(nfc1), wspec(nbfc1), wspec(nfc2), wspec(nbfc2)],
        out_specs=pl.BlockSpec((NB, D), lambda i: (i, 0)),
        out_shape=jax.ShapeDtypeStruct((N, D), jnp.float32),
    )(s, ve, ei0i, nt, wout_n, bout_n, lnf[0], lnf[1],
      nfc1, nbfc1, nfc2, nbfc2)

    g23 = _sc_gather(nt2, idx)
    g2 = g23[:E]
    g3 = g23[E:]

    lnn2 = ln["node_attn_ln2"]
    lne2 = ln["edge_attn_ln2"]
    qkv = pl.pallas_call(
        _eqkv_body,
        grid=(E // EB,),
        in_specs=[_eblk(), _eblk(), _eblk(),
                  wspec(lne2[0]), wspec(lne2[1]), wspec(lnn2[0]), wspec(lnn2[1]),
                  wspec(weqkv_e), wspec(beqkv_e), wspec(wnqkv_e), wspec(bnqkv_e)],
        out_specs=_eblk(3 * D),
        out_shape=jax.ShapeDtypeStruct((E, 3 * D), jnp.float32),
    )(et, g2, g3, lne2[0], lne2[1], lnn2[0], lnn2[1],
      weqkv_e, beqkv_e, wnqkv_e, bnqkv_e)

    lnfe = ln["edge_ffn_ln"]
    rowspec = lambda cols: pl.BlockSpec((EB, cols), lambda i, c: (i, 0))
    w2 = lambda a: pl.BlockSpec(a.shape, lambda i, c: (0, 0))
    et2 = pl.pallas_call(
        _eattn_body,
        grid=(E // EB, E // CB),
        in_specs=[rowspec(3 * D),
                  pl.BlockSpec((CB, 3 * D), lambda i, c: (c, 0)),
                  pl.BlockSpec((1, CB), lambda i, c: (0, c)),
                  pl.BlockSpec((1, CB), lambda i, c: (0, c)),
                  rowspec(1), rowspec(1), rowspec(D),
                  w2(wout_e), w2(bout_e), w2(lnfe[0]), w2(lnfe[1]),
                  w2(efc1), w2(ebfc1), w2(efc2), w2(ebfc2)],
        out_specs=rowspec(D),
        out_shape=jax.ShapeDtypeStruct((E, D), jnp.float32),
        scratch_shapes=[pltpu.VMEM((EB, H), jnp.float32),
                        pltpu.VMEM((EB, H), jnp.float32),
                        pltpu.VMEM((EB, D), jnp.float32)],
    )(qkv, qkv, ei0r, ei1r, ei0c, ei1c, et,
      wout_e, bout_e, lnfe[0], lnfe[1], efc1, ebfc1, efc2, ebfc2)

    return nt2, et2


# speedup vs baseline: 4.7733x; 1.7180x over previous
import functools

import jax
import jax.numpy as jnp
import numpy as np
from jax import lax
from jax.experimental import pallas as pl
from jax.experimental.pallas import tpu as pltpu
from jax.experimental.pallas import tpu_sc as plsc

N = 1024
E = 4096
D = 128
H = 8
DH = D // H
HID = 256
SCALE = 1.0 / np.sqrt(DH)
NEG = -1e30

EB = 512
NB = 256
CB = 512


def _sc_gather(table, idx):
    B = idx.shape[0]
    Dt = table.shape[1]
    info = plsc.get_sparse_core_info()
    nc = info.num_cores
    nw = nc * info.num_subcores
    b_per_w = B // nw
    chunk = min(b_per_w, 128)
    n_chunks = b_per_w // chunk
    mesh = plsc.VectorSubcoreMesh(core_axis_name="c", subcore_axis_name="s")

    @functools.partial(
        pl.kernel, mesh=mesh,
        out_type=jax.ShapeDtypeStruct((B, Dt), jnp.float32),
        scratch_types=[
            pltpu.VMEM((n_chunks, chunk), jnp.int32),
            pltpu.VMEM((b_per_w, Dt), jnp.float32),
            pltpu.SemaphoreType.DMA,
        ],
    )
    def k(table_hbm, idx_hbm, out_hbm, idx_v, rows_v, sem):
        wid = lax.axis_index("s") * nc + lax.axis_index("c")
        base = wid * b_per_w
        for j in range(n_chunks):
            pltpu.sync_copy(idx_hbm.at[pl.ds(base + j * chunk, chunk)],
                            idx_v.at[j])
            pltpu.async_copy(table_hbm.at[idx_v.at[j]],
                             rows_v.at[pl.ds(j * chunk, chunk)], sem).wait()
        pltpu.sync_copy(rows_v, out_hbm.at[pl.ds(base, b_per_w)])

    return k(table, idx)


def _ln(x, g, b):
    m = jnp.mean(x, axis=-1, keepdims=True)
    v = jnp.mean((x - m) ** 2, axis=-1, keepdims=True)
    return (x - m) / jnp.sqrt(v + 1e-5) * g + b


def _dot(a, b):
    return jnp.dot(a, b, preferred_element_type=jnp.float32)


def _rep_heads(x, rows):
    return jnp.concatenate(
        [jnp.broadcast_to(x[:, h:h + 1], (rows, DH)) for h in range(H)], axis=1)


def _nprep_body(et_ref, g0_ref, g1_ref, lne_g, lne_b, lnn_g, lnn_b,
                weqkv, beqkv, wnq, bnq, wnkv, bnkv, s_ref, ve_ref):
    xe = _ln(et_ref[...], lne_g[...], lne_b[...])
    eqkv = _dot(xe, weqkv[...]) + beqkv[...]
    x0 = _ln(g0_ref[...], lnn_g[...], lnn_b[...])
    x1 = _ln(g1_ref[...], lnn_g[...], lnn_b[...])
    q = eqkv[:, :D] + _dot(x0, wnq[...]) + bnq[...]
    kv = eqkv[:, D:] + _dot(x1, wnkv[...]) + bnkv[...]
    ve_ref[...] = kv[:, D:]
    qk = q * kv[:, :D]
    cols = [jnp.sum(qk[:, h * DH:(h + 1) * DH], axis=1, keepdims=True)
            for h in range(H)]
    s_ref[...] = jnp.concatenate(cols, axis=1) * SCALE


def _nagg_body(s_ref, ve_ref, ei0_ref, nt_ref, wout, bout, lnf_g, lnf_b,
               fc1, bfc1, fc2, bfc2, out_ref):
    i = pl.program_id(0)
    s = s_ref[...]
    m = jnp.max(s, axis=0, keepdims=True)
    e = jnp.exp(s - m)
    ev = _rep_heads(e, E) * ve_ref[...]
    rows = lax.broadcasted_iota(jnp.int32, (NB, E), 0) + (i * NB)
    oh = jnp.where(rows == ei0_ref[...], 1.0, 0.0)
    den = _dot(oh, e)
    acc = _dot(oh, ev)
    upd = acc / (_rep_heads(den, NB) + 1e-16)
    nt1 = _dot(upd, wout[...]) + bout[...] + nt_ref[...]
    x = _ln(nt1, lnf_g[...], lnf_b[...])
    h1 = jnp.maximum(_dot(x, fc1[...]) + bfc1[...], 0.0)
    out_ref[...] = _dot(h1, fc2[...]) + bfc2[...] + nt1


def _eqkv_body(et_ref, g2_ref, g3_ref, lne_g, lne_b, lnn_g, lnn_b,
               weqkv, beqkv, wnqkv, bnqkv, out_ref):
    xe = _ln(et_ref[...], lne_g[...], lne_b[...])
    base = _dot(xe, weqkv[...]) + beqkv[...]
    x2 = _ln(g2_ref[...], lnn_g[...], lnn_b[...])
    x3 = _ln(g3_ref[...], lnn_g[...], lnn_b[...])
    out_ref[...] = base + _dot(x2 + x3, wnqkv[...]) + 2.0 * bnqkv[...]


def _eattn_body(qkv_blk, kv_chunk, ei0r, ei1r, ei0c, ei1c, et_ref,
                wout, bout, lnf_g, lnf_b, fc1, bfc1, fc2, bfc2, out_ref,
                den_s, acc_s):
    c = pl.program_id(1)
    nc = pl.num_programs(1)

    @pl.when(c == 0)
    def _init():
        den_s[...] = jnp.zeros((EB, H), jnp.float32)
        acc_s[...] = jnp.zeros((EB, D), jnp.float32)

    qall = qkv_blk[...][:, :D]
    kc = kv_chunk[...][:, D:2 * D]
    vc = kv_chunk[...][:, 2 * D:]
    a0 = ei0c[...]
    a1 = ei1c[...]
    b0 = ei0r[...]
    b1 = ei1r[...]
    nb = (a0 == b0) | (a0 == b1) | (a1 == b0) | (a1 == b1)
    neg = jnp.where(nb, 0.0, NEG)
    lane = lax.broadcasted_iota(jnp.int32, (1, D), 1) // DH
    accv = jnp.zeros((EB, D), jnp.float32)
    dens = []
    for h in range(H):
        mh = lane == h
        kh = jnp.where(mh, kc, 0.0)
        vh = jnp.where(mh, vc, 0.0).astype(jnp.bfloat16)
        sh = lax.dot_general(qall, kh, (((1,), (1,)), ((), ())),
                             preferred_element_type=jnp.float32)
        ph = jnp.exp(sh * SCALE + neg)
        dens.append(jnp.sum(ph, axis=1, keepdims=True))
        accv = accv + _dot(ph.astype(jnp.bfloat16), vh)
    den_s[...] = den_s[...] + jnp.concatenate(dens, axis=1)
    acc_s[...] = acc_s[...] + accv

    @pl.when(c == nc - 1)
    def _epilogue():
        upd = acc_s[...] / (_rep_heads(den_s[...], EB) + 1e-16)
        et1 = _dot(upd, wout[...]) + bout[...] + et_ref[...]
        x = _ln(et1, lnf_g[...], lnf_b[...])
        h1 = jnp.maximum(_dot(x, fc1[...]) + bfc1[...], 0.0)
        out_ref[...] = _dot(h1, fc2[...]) + bfc2[...] + et1


def _row(v):
    return v.reshape(1, -1)


def _full(shape):
    return pl.BlockSpec(shape, lambda i: (0, 0))


def _eblk(cols=D):
    return pl.BlockSpec((EB, cols), lambda i: (i, 0))


def kernel(node_tensors, edge_tensors, edge_index, params):
    p = params
    nt = node_tensors
    et = edge_tensors
    ei0 = edge_index[0]
    ei1 = edge_index[1]
    idx = jnp.concatenate([ei0, ei1])
    ei0i = _row(ei0)
    ei0r = _row(ei0.astype(jnp.float32))
    ei1r = _row(ei1.astype(jnp.float32))
    ei0c = ei0.astype(jnp.float32).reshape(E, 1)
    ei1c = ei1.astype(jnp.float32).reshape(E, 1)

    nm = p["node_mha"]
    em = p["edge_mha"]
    weqkv_n = jnp.concatenate(
        [nm["Weq"]["W"].T, nm["Wek"]["W"].T, nm["Wev"]["W"].T], axis=1)
    beqkv_n = _row(jnp.concatenate(
        [nm["Weq"]["b"], nm["Wek"]["b"], nm["Wev"]["b"]]))
    wnq_n = nm["Wnq"]["W"].T
    bnq_n = _row(nm["Wnq"]["b"])
    wnkv_n = jnp.concatenate([nm["Wnk"]["W"].T, nm["Wnv"]["W"].T], axis=1)
    bnkv_n = _row(jnp.concatenate([nm["Wnk"]["b"], nm["Wnv"]["b"]]))
    wout_n = nm["out"]["W"].T
    bout_n = _row(nm["out"]["b"])
    weqkv_e = jnp.concatenate(
        [em["Weq"]["W"].T, em["Wek"]["W"].T, em["Wev"]["W"].T], axis=1)
    beqkv_e = _row(jnp.concatenate(
        [em["Weq"]["b"], em["Wek"]["b"], em["Wev"]["b"]]))
    wnqkv_e = jnp.concatenate(
        [em["Wnq"]["W"].T, em["Wnk"]["W"].T, em["Wnv"]["W"].T], axis=1)
    bnqkv_e = _row(jnp.concatenate(
        [em["Wnq"]["b"], em["Wnk"]["b"], em["Wnv"]["b"]]))
    wout_e = em["out"]["W"].T
    bout_e = _row(em["out"]["b"])
    nfc1 = p["node_ffn_fc1"]["W"].T
    nbfc1 = _row(p["node_ffn_fc1"]["b"])
    nfc2 = p["node_ffn_fc2"]["W"].T
    nbfc2 = _row(p["node_ffn_fc2"]["b"])
    efc1 = p["edge_ffn_fc1"]["W"].T
    ebfc1 = _row(p["edge_ffn_fc1"]["b"])
    efc2 = p["edge_ffn_fc2"]["W"].T
    ebfc2 = _row(p["edge_ffn_fc2"]["b"])
    ln = {k: (_row(v["g"]), _row(v["b"]))
          for k, v in p.items() if k.endswith("ln") or "_ln" in k}

    g01 = _sc_gather(nt, idx)
    g0 = g01[:E]
    g1 = g01[E:]

    wspec = lambda a: _full(a.shape)
    lnn1 = ln["node_attn_ln1"]
    lne1 = ln["edge_attn_ln1"]
    s, ve = pl.pallas_call(
        _nprep_body,
        grid=(E // EB,),
        in_specs=[_eblk(), _eblk(), _eblk(),
                  wspec(lne1[0]), wspec(lne1[1]), wspec(lnn1[0]), wspec(lnn1[1]),
                  wspec(weqkv_n), wspec(beqkv_n), wspec(wnq_n), wspec(bnq_n),
                  wspec(wnkv_n), wspec(bnkv_n)],
        out_specs=[_eblk(H), _eblk(D)],
        out_shape=[jax.ShapeDtypeStruct((E, H), jnp.float32),
                   jax.ShapeDtypeStruct((E, D), jnp.float32)],
    )(et, g0, g1, lne1[0], lne1[1], lnn1[0], lnn1[1],
      weqkv_n, beqkv_n, wnq_n, bnq_n, wnkv_n, bnkv_n)

    lnf = ln["node_ffn_ln"]
    nt2 = pl.pallas_call(
        _nagg_body,
        grid=(N // NB,),
        in_specs=[_full((E, H)), _full((E, D)), _full((1, E)),
                  pl.BlockSpec((NB, D), lambda i: (i, 0)),
                  wspec(wout_n), wspec(bout_n), wspec(lnf[0]), wspec(lnf[1]),
                  wspec(nfc1), wspec(nbfc1), wspec(nfc2), wspec(nbfc2)],
        out_specs=pl.BlockSpec((NB, D), lambda i: (i, 0)),
        out_shape=jax.ShapeDtypeStruct((N, D), jnp.float32),
    )(s, ve, ei0i, nt, wout_n, bout_n, lnf[0], lnf[1],
      nfc1, nbfc1, nfc2, nbfc2)

    g23 = _sc_gather(nt2, idx)
    g2 = g23[:E]
    g3 = g23[E:]

    lnn2 = ln["node_attn_ln2"]
    lne2 = ln["edge_attn_ln2"]
    qkv = pl.pallas_call(
        _eqkv_body,
        grid=(E // EB,),
        in_specs=[_eblk(), _eblk(), _eblk(),
                  wspec(lne2[0]), wspec(lne2[1]), wspec(lnn2[0]), wspec(lnn2[1]),
                  wspec(weqkv_e), wspec(beqkv_e), wspec(wnqkv_e), wspec(bnqkv_e)],
        out_specs=_eblk(3 * D),
        out_shape=jax.ShapeDtypeStruct((E, 3 * D), jnp.float32),
    )(et, g2, g3, lne2[0], lne2[1], lnn2[0], lnn2[1],
      weqkv_e, beqkv_e, wnqkv_e, bnqkv_e)

    lnfe = ln["edge_ffn_ln"]
    rowspec = lambda cols: pl.BlockSpec((EB, cols), lambda i, c: (i, 0))
    w2 = lambda a: pl.BlockSpec(a.shape, lambda i, c: (0, 0))
    et2 = pl.pallas_call(
        _eattn_body,
        grid=(E // EB, E // CB),
        in_specs=[rowspec(3 * D),
                  pl.BlockSpec((CB, 3 * D), lambda i, c: (c, 0)),
                  pl.BlockSpec((1, CB), lambda i, c: (0, c)),
                  pl.BlockSpec((1, CB), lambda i, c: (0, c)),
                  rowspec(1), rowspec(1), rowspec(D),
                  w2(wout_e), w2(bout_e), w2(lnfe[0]), w2(lnfe[1]),
                  w2(efc1), w2(ebfc1), w2(efc2), w2(ebfc2)],
        out_specs=rowspec(D),
        out_shape=jax.ShapeDtypeStruct((E, D), jnp.float32),
        scratch_shapes=[pltpu.VMEM((EB, H), jnp.float32),
                        pltpu.VMEM((EB, D), jnp.float32)],
    )(qkv, qkv, ei0r, ei1r, ei0c, ei1c, et,
      wout_e, bout_e, lnfe[0], lnfe[1], efc1, ebfc1, efc2, ebfc2)

    return nt2, et2
